# baseline (speedup 1.0000x reference)
import functools

import jax
import jax.numpy as jnp
from jax import lax
from jax.experimental import pallas as pl
from jax.experimental.pallas import tpu as pltpu
from jax.experimental.pallas import tpu_sc as plsc

A = 46
LANES = 16
NC, NS = 2, 16
NW = NC * NS


def _sc_body(row0, rows_per_w, chunk_rows, n_chunks,
             clear_hbm, shifts_hbm, out_hbm, in_v, out_v, sh_v):
    wid = lax.axis_index("s") * NC + lax.axis_index("c")
    wrow0 = row0 + wid * rows_per_w
    chunk_elems = chunk_rows * A
    vregs = chunk_elems // LANES
    iota = lax.iota(jnp.int32, LANES)

    def do_chunk(c, _):
        crow = wrow0 + c * chunk_rows
        pltpu.sync_copy(clear_hbm.at[pl.ds(crow, chunk_rows), :], in_v)
        pltpu.sync_copy(shifts_hbm.at[pl.ds(crow, chunk_rows)], sh_v)

        @plsc.parallel_loop(0, vregs, 1, unroll=8)
        def _(i):
            p = i * LANES + iota
            row = lax.div(p, A)
            a = p - row * A
            s = plsc.load_gather(sh_v, [row])
            col = a - s
            col = jnp.where(col < 0, col + A, col)
            val = plsc.load_gather(in_v, [row, col])
            plsc.store_scatter(out_v, [row, a], val)

        pltpu.sync_copy(out_v, out_hbm.at[pl.ds(crow - row0, chunk_rows), :])
        return 0

    lax.fori_loop(0, n_chunks, do_chunk, 0)


@functools.partial(jax.jit,
                   static_argnames=("row0", "rows_per_w", "chunk_rows", "n_chunks"))
def _sc_call(clear2, shifts_flat, row0, rows_per_w, chunk_rows, n_chunks):
    n_rows = rows_per_w * NW
    body = functools.partial(_sc_body, row0, rows_per_w, chunk_rows, n_chunks)
    return pl.kernel(
        body,
        out_type=jax.ShapeDtypeStruct((n_rows, A), clear2.dtype),
        mesh=plsc.VectorSubcoreMesh(core_axis_name="c", subcore_axis_name="s"),
        scratch_types=[
            pltpu.VMEM((chunk_rows, A), jnp.float32),
            pltpu.VMEM((chunk_rows, A), jnp.float32),
            pltpu.VMEM((chunk_rows,), jnp.int32),
        ],
        compiler_params=pltpu.CompilerParams(
            needs_layout_passes=False, use_tc_tiling_on_sc=True),
    )(clear2, shifts_flat)


def kernel(clear, shifts):
    b, l, a = clear.shape
    n_rows = b * l
    clear2 = clear.reshape(n_rows, a)
    shifts_flat = shifts.reshape(-1)
    rows_per_w = n_rows // NW
    chunk_rows = 256
    out = _sc_call(clear2, shifts_flat, 0, rows_per_w, chunk_rows,
                   rows_per_w // chunk_rows)
    return out.reshape(b, l, a)


# speedup vs baseline: 1.2184x; 1.2184x over previous
import functools

import jax
import jax.numpy as jnp
from jax import lax
from jax.experimental import pallas as pl
from jax.experimental.pallas import tpu as pltpu
from jax.experimental.pallas import tpu_sc as plsc

A = 46
LANES = 16
NC, NS = 2, 16
NW = NC * NS


def _sc_body(row0, rows_per_w, chunk_rows, n_chunks,
             clear_hbm, shifts_hbm, out_hbm, in_v, out_v, sh_v):
    wid = lax.axis_index("s") * NC + lax.axis_index("c")
    wrow0 = row0 + wid * rows_per_w
    chunk_elems = chunk_rows * A
    vregs = chunk_elems // LANES
    iota = lax.iota(jnp.int32, LANES)

    def do_chunk(c, _):
        crow = wrow0 + c * chunk_rows
        pltpu.sync_copy(clear_hbm.at[pl.ds(crow, chunk_rows), :], in_v)
        pltpu.sync_copy(shifts_hbm.at[pl.ds(crow, chunk_rows)], sh_v)

        @plsc.parallel_loop(0, vregs, 1, unroll=8)
        def _(i):
            p = i * LANES + iota
            row = lax.div(p, A)
            a = p - row * A
            s = plsc.load_gather(sh_v, [row])
            col = a - s
            col = jnp.where(col < 0, col + A, col)
            val = plsc.load_gather(in_v, [row, col])
            plsc.store_scatter(out_v, [row, a], val)

        pltpu.sync_copy(out_v, out_hbm.at[pl.ds(crow - row0, chunk_rows), :])
        return 0

    lax.fori_loop(0, n_chunks, do_chunk, 0)


@functools.partial(jax.jit,
                   static_argnames=("row0", "rows_per_w", "chunk_rows", "n_chunks"))
def _sc_call(clear2, shifts_flat, row0, rows_per_w, chunk_rows, n_chunks):
    n_rows = rows_per_w * NW
    body = functools.partial(_sc_body, row0, rows_per_w, chunk_rows, n_chunks)
    return pl.kernel(
        body,
        out_type=jax.ShapeDtypeStruct((n_rows, A), clear2.dtype),
        mesh=plsc.VectorSubcoreMesh(core_axis_name="c", subcore_axis_name="s"),
        scratch_types=[
            pltpu.VMEM((chunk_rows, A), jnp.float32),
            pltpu.VMEM((chunk_rows, A), jnp.float32),
            pltpu.VMEM((chunk_rows,), jnp.int32),
        ],
        compiler_params=pltpu.CompilerParams(
            needs_layout_passes=False, use_tc_tiling_on_sc=True),
    )(clear2, shifts_flat)


def _tc_body(x_ref, s_ref, o_ref):
    x = x_ref[...]
    s2 = s_ref[...]
    g, l = s2.shape
    x3 = x.reshape(g, l, A)
    s3 = s2.reshape(g, l, 1)
    lane = lax.broadcasted_iota(jnp.int32, (g, l, A), 2)
    col = lane - s3
    col = jnp.where(col < 0, col + A, col)
    o_ref[...] = jnp.take_along_axis(x3, col, axis=2).reshape(g * l, A)


@functools.partial(jax.jit, static_argnames=("g_rows", "b_tc"))
def _tc_call(clear2, shifts, g_rows, b_tc):
    n_rows, a = clear2.shape
    l = shifts.shape[1]
    return pl.pallas_call(
        _tc_body,
        grid=(b_tc // g_rows,),
        in_specs=[
            pl.BlockSpec((g_rows * l, a), lambda i: (i, 0)),
            pl.BlockSpec((g_rows, l), lambda i: (i, 0)),
        ],
        out_specs=pl.BlockSpec((g_rows * l, a), lambda i: (i, 0)),
        out_shape=jax.ShapeDtypeStruct((n_rows, a), clear2.dtype),
        compiler_params=pltpu.CompilerParams(
            dimension_semantics=("parallel",)),
    )(clear2, shifts)


ROWS_TC = 460800
SC_CHUNK = 200


def kernel(clear, shifts):
    b, l, a = clear.shape
    n_rows = b * l
    clear2 = clear.reshape(n_rows, a)
    shifts_flat = shifts.reshape(-1)
    rows_sc = n_rows - ROWS_TC
    rpw = rows_sc // NW
    tc_out = _tc_call(clear2, shifts, 64, ROWS_TC // l)
    sc_out = _sc_call(clear2, shifts_flat, ROWS_TC, rpw, SC_CHUNK,
                      rpw // SC_CHUNK)
    out = lax.dynamic_update_slice(tc_out, sc_out, (ROWS_TC, 0))
    return out.reshape(b, l, a)


# speedup vs baseline: 1.5345x; 1.2595x over previous
import functools

import jax
import jax.numpy as jnp
from jax import lax
from jax.experimental import pallas as pl
from jax.experimental.pallas import tpu as pltpu
from jax.experimental.pallas import tpu_sc as plsc

A = 46
LANES = 16
NC, NS = 2, 16
NW = NC * NS


def _sc_body(row0, rows_per_w, chunk_rows, n_chunks,
             clear_hbm, shifts_hbm, out_hbm,
             in_v0, in_v1, out_v0, out_v1, sh_v0, sh_v1,
             in_s0, in_s1, out_s0, out_s1, sh_s0, sh_s1):
    wid = lax.axis_index("s") * NC + lax.axis_index("c")
    wrow0 = row0 + wid * rows_per_w
    chunk_elems = chunk_rows * A
    vregs = chunk_elems // LANES
    iota = lax.iota(jnp.int32, LANES)
    in_v = (in_v0, in_v1)
    out_v = (out_v0, out_v1)
    sh_v = (sh_v0, sh_v1)
    in_s = (in_s0, in_s1)
    out_s = (out_s0, out_s1)
    sh_s = (sh_s0, sh_s1)

    def start_in(c, ph):
        crow = wrow0 + c * chunk_rows
        pltpu.async_copy(clear_hbm.at[pl.ds(crow, chunk_rows), :],
                         in_v[ph], in_s[ph])
        pltpu.async_copy(shifts_hbm.at[pl.ds(crow, chunk_rows)],
                         sh_v[ph], sh_s[ph])

    def wait_in(c, ph):
        crow = wrow0 + c * chunk_rows
        pltpu.make_async_copy(clear_hbm.at[pl.ds(crow, chunk_rows), :],
                              in_v[ph], in_s[ph]).wait()
        pltpu.make_async_copy(shifts_hbm.at[pl.ds(crow, chunk_rows)],
                              sh_v[ph], sh_s[ph]).wait()

    def start_out(c, ph):
        crow = wrow0 + c * chunk_rows
        pltpu.async_copy(out_v[ph],
                         out_hbm.at[pl.ds(crow - row0, chunk_rows), :],
                         out_s[ph])

    def wait_out(c, ph):
        crow = wrow0 + c * chunk_rows
        pltpu.make_async_copy(out_v[ph],
                              out_hbm.at[pl.ds(crow - row0, chunk_rows), :],
                              out_s[ph]).wait()

    def compute(ph):
        @plsc.parallel_loop(0, vregs, 1, unroll=8)
        def _(i):
            p = i * LANES + iota
            row = lax.div(p, A)
            a = p - row * A
            s = plsc.load_gather(sh_v[ph], [row])
            col = a - s
            col = jnp.where(col < 0, col + A, col)
            val = plsc.load_gather(in_v[ph], [row, col])
            plsc.store_scatter(out_v[ph], [row, a], val)

    start_in(0, 0)

    def pair_body(c2, _):
        c = 2 * c2
        start_in(c + 1, 1)
        wait_in(c, 0)

        @pl.when(c2 > 0)
        def _():
            wait_out(c - 2, 0)

        compute(0)
        start_out(c, 0)
        @pl.when(c + 2 < n_chunks)
        def _():
            start_in(c + 2, 0)

        wait_in(c + 1, 1)

        @pl.when(c2 > 0)
        def _():
            wait_out(c - 1, 1)

        compute(1)
        start_out(c + 1, 1)
        return 0

    lax.fori_loop(0, n_chunks // 2, pair_body, 0)
    wait_out(n_chunks - 2, 0)
    wait_out(n_chunks - 1, 1)


@functools.partial(jax.jit,
                   static_argnames=("row0", "rows_per_w", "chunk_rows", "n_chunks"))
def _sc_call(clear2, shifts_flat, row0, rows_per_w, chunk_rows, n_chunks):
    n_rows = rows_per_w * NW
    body = functools.partial(_sc_body, row0, rows_per_w, chunk_rows, n_chunks)
    return pl.kernel(
        body,
        out_type=jax.ShapeDtypeStruct((n_rows, A), clear2.dtype),
        mesh=plsc.VectorSubcoreMesh(core_axis_name="c", subcore_axis_name="s"),
        scratch_types=[
            pltpu.VMEM((chunk_rows, A), jnp.float32),
            pltpu.VMEM((chunk_rows, A), jnp.float32),
            pltpu.VMEM((chunk_rows, A), jnp.float32),
            pltpu.VMEM((chunk_rows, A), jnp.float32),
            pltpu.VMEM((chunk_rows,), jnp.int32),
            pltpu.VMEM((chunk_rows,), jnp.int32),
            pltpu.SemaphoreType.DMA,
            pltpu.SemaphoreType.DMA,
            pltpu.SemaphoreType.DMA,
            pltpu.SemaphoreType.DMA,
            pltpu.SemaphoreType.DMA,
            pltpu.SemaphoreType.DMA,
        ],
        compiler_params=pltpu.CompilerParams(
            needs_layout_passes=False, use_tc_tiling_on_sc=True),
    )(clear2, shifts_flat)


def kernel(clear, shifts):
    b, l, a = clear.shape
    n_rows = b * l
    clear2 = clear.reshape(n_rows, a)
    shifts_flat = shifts.reshape(-1)
    rows_per_w = n_rows // NW
    chunk_rows = 200
    out = _sc_call(clear2, shifts_flat, 0, rows_per_w, chunk_rows,
                   rows_per_w // chunk_rows)
    return out.reshape(b, l, a)
